# Initial kernel scaffold; baseline (speedup 1.0000x reference)
import jax
import jax.numpy as jnp
from jax.experimental import pallas as pl


def kernel(z, achs, anchor_att_ret, Wq, Wk, Wd, bd):
    raise NotImplementedError("write your pallas kernel here")


# speedup vs baseline: 2.0988x; 2.0988x over previous
import functools

import jax
import jax.numpy as jnp
from jax import lax
from jax.experimental import pallas as pl
from jax.experimental.pallas import tpu as pltpu

B, K, A, D = 128, 20, 64, 512
TOPK = K
BB = 8

_NEG = -3.0e38
_BIGI = 10_000_000


def _tc_body(z_ref, achs_ref, ret_ref, wq_ref, wk_ref, wd_ref, bd_ref,
             zout_ref, att_ref, idx_ref):
    zb = z_ref[...]
    ab = achs_ref[...]
    wq = wq_ref[...]
    wk = wk_ref[...]
    wd = wd_ref[...]
    bd = bd_ref[...]

    q = jnp.dot(ab.reshape(BB * A, D), wq,
                preferred_element_type=jnp.float32).reshape(BB, A, D)
    kk = jnp.dot(zb.reshape(BB * K, D), wk,
                 preferred_element_type=jnp.float32).reshape(BB, K, D)
    disp = (jnp.dot(ab.reshape(BB * A, D), wd,
                    preferred_element_type=jnp.float32)
            + bd).reshape(BB, A, D)

    scale = 1.0 / (D ** 0.5)
    logits_list = []
    for b in range(BB):
        lg = lax.dot_general(q[b], kk[b], (((1,), (1,)), ((), ())),
                             preferred_element_type=jnp.float32)
        logits_list.append(lg * scale)
    logits = jnp.stack(logits_list, axis=0)

    m = jnp.max(logits, axis=2, keepdims=True)
    lse = jnp.log(jnp.sum(jnp.exp(logits - m), axis=2, keepdims=True)) + m
    att_log = logits - lse

    ret = ret_ref[...]
    y = (att_log + ret[:, None, :]).reshape(BB, A * K)

    iota = lax.broadcasted_iota(jnp.int32, (BB, A * K), 1)
    vals = []
    idxs = []
    x = y
    for _ in range(TOPK):
        mx = jnp.max(x, axis=1, keepdims=True)
        im = jnp.min(jnp.where(x == mx, iota, _BIGI), axis=1,
                     keepdims=True)
        vals.append(mx)
        idxs.append(im)
        x = jnp.where(iota == im, _NEG, x)
    val = jnp.concatenate(vals, axis=1)
    fidx = jnp.concatenate(idxs, axis=1)

    a_idx = fidx // K
    k_idx = fidx - a_idx * K
    idx_ref[...] = k_idx * A + a_idx

    iota_k = lax.broadcasted_iota(jnp.int32, (TOPK, K), 1)
    iota_a = lax.broadcasted_iota(jnp.int32, (TOPK, A), 1)
    zn_list = []
    for b in range(BB):
        oh_k = (k_idx[b][:, None] == iota_k).astype(jnp.float32)
        oh_a = (a_idx[b][:, None] == iota_a).astype(jnp.float32)
        zn = (jnp.dot(oh_k, zb[b], preferred_element_type=jnp.float32)
              + jnp.dot(oh_a, disp[b], preferred_element_type=jnp.float32))
        zn_list.append(zn)
    znew = jnp.stack(zn_list, axis=0)

    mu = jnp.mean(znew, axis=2, keepdims=True)
    var = jnp.mean((znew - mu) ** 2, axis=2, keepdims=True)
    zout_ref[...] = (znew - mu) / jnp.sqrt(var + 1e-5)

    att_ref[...] = val - jnp.max(val, axis=1, keepdims=True)


@jax.jit
def _run(z, achs, ret2d, Wq, Wk, Wd, bd2d):
    grid = (B // BB,)
    out_shapes = (
        jax.ShapeDtypeStruct((B, TOPK, D), jnp.float32),
        jax.ShapeDtypeStruct((B, TOPK), jnp.float32),
        jax.ShapeDtypeStruct((B, TOPK), jnp.int32),
    )
    zb_spec = pl.BlockSpec((BB, K, D), lambda i: (i, 0, 0))
    ab_spec = pl.BlockSpec((BB, A, D), lambda i: (i, 0, 0))
    ret_spec = pl.BlockSpec((BB, K), lambda i: (i, 0))
    w_spec = pl.BlockSpec((D, D), lambda i: (0, 0))
    bd_spec = pl.BlockSpec((1, D), lambda i: (0, 0))
    out_specs = (
        pl.BlockSpec((BB, TOPK, D), lambda i: (i, 0, 0)),
        pl.BlockSpec((BB, TOPK), lambda i: (i, 0)),
        pl.BlockSpec((BB, TOPK), lambda i: (i, 0)),
    )
    return pl.pallas_call(
        _tc_body,
        grid=grid,
        in_specs=[zb_spec, ab_spec, ret_spec, w_spec, w_spec, w_spec,
                  bd_spec],
        out_specs=out_specs,
        out_shape=out_shapes,
    )(z, achs, ret2d, Wq, Wk, Wd, bd2d)


def kernel(z, achs, anchor_att_ret, Wq, Wk, Wd, bd):
    ret2d = anchor_att_ret.reshape(B, K)
    bd2d = bd.reshape(1, D)
    z_out, att, idx = _run(z, achs, ret2d, Wq, Wk, Wd, bd2d)
    return (z_out, att.reshape(B, TOPK, 1), idx)


# speedup vs baseline: 3.7915x; 1.8065x over previous
import functools

import jax
import jax.numpy as jnp
from jax import lax
from jax.experimental import pallas as pl
from jax.experimental.pallas import tpu as pltpu

B, K, A, D = 128, 20, 64, 512
TOPK = K
BB = 32

_NEG = -3.0e38


def _tc_body(z_ref, achs_ref, ret_ref, wqd_ref, wk_ref, bd_ref,
             zout_ref, att_ref, idx_ref):
    zb = z_ref[...]
    ab = achs_ref[...]
    wqd = wqd_ref[...]
    wk = wk_ref[...]
    bd = bd_ref[...]

    qd = jnp.dot(ab.reshape(BB * A, D), wqd,
                 preferred_element_type=jnp.float32)
    q = qd[:, :D].reshape(BB, A, D)
    disp = (qd[:, D:] + bd).reshape(BB, A, D)
    kk = jnp.dot(zb.reshape(BB * K, D), wk,
                 preferred_element_type=jnp.float32).reshape(BB, K, D)

    scale = 1.0 / (D ** 0.5)
    logits = lax.dot_general(q, kk, (((2,), (2,)), ((0,), (0,))),
                             preferred_element_type=jnp.float32) * scale

    m = jnp.max(logits, axis=2, keepdims=True)
    lse = jnp.log(jnp.sum(jnp.exp(logits - m), axis=2, keepdims=True)) + m
    att_log = logits - lse

    ret = ret_ref[...]
    y = (att_log + ret[:, None, :]).reshape(BB, A * K)

    iota = lax.broadcasted_iota(jnp.int32, (BB, A * K), 1)
    vals = []
    idxs = []
    x = y
    for t in range(TOPK):
        mx = jnp.max(x, axis=1, keepdims=True)
        im = jnp.min(jnp.where(x == mx, iota, 10_000_000), axis=1,
                     keepdims=True)
        vals.append(mx)
        idxs.append(im)
        if t + 1 < TOPK:
            x = jnp.where(iota == im, _NEG, x)
    val = jnp.concatenate(vals, axis=1)
    fidx = jnp.concatenate(idxs, axis=1)

    a_idx = fidx // K
    k_idx = fidx - a_idx * K
    idx_ref[...] = k_idx * A + a_idx

    iota_ka = lax.broadcasted_iota(jnp.int32, (TOPK, K + A), 1)
    comb_idx = jnp.concatenate(
        [k_idx[:, :, None], (a_idx + K)[:, :, None]], axis=2)
    oh = ((comb_idx[:, :, 0:1] == iota_ka[None])
          | (comb_idx[:, :, 1:2] == iota_ka[None])).astype(jnp.float32)
    cat = jnp.concatenate([zb, disp], axis=1)
    znew = lax.dot_general(oh, cat, (((2,), (1,)), ((0,), (0,))),
                           preferred_element_type=jnp.float32)

    mu = jnp.mean(znew, axis=2, keepdims=True)
    var = jnp.mean((znew - mu) ** 2, axis=2, keepdims=True)
    zout_ref[...] = (znew - mu) / jnp.sqrt(var + 1e-5)

    att_ref[...] = val - val[:, 0:1]


@jax.jit
def _run(z, achs, ret2d, Wqd, Wk, bd2d):
    grid = (B // BB,)
    out_shapes = (
        jax.ShapeDtypeStruct((B, TOPK, D), jnp.float32),
        jax.ShapeDtypeStruct((B, TOPK), jnp.float32),
        jax.ShapeDtypeStruct((B, TOPK), jnp.int32),
    )
    zb_spec = pl.BlockSpec((BB, K, D), lambda i: (i, 0, 0))
    ab_spec = pl.BlockSpec((BB, A, D), lambda i: (i, 0, 0))
    ret_spec = pl.BlockSpec((BB, K), lambda i: (i, 0))
    wqd_spec = pl.BlockSpec((D, 2 * D), lambda i: (0, 0))
    wk_spec = pl.BlockSpec((D, D), lambda i: (0, 0))
    bd_spec = pl.BlockSpec((1, D), lambda i: (0, 0))
    out_specs = (
        pl.BlockSpec((BB, TOPK, D), lambda i: (i, 0, 0)),
        pl.BlockSpec((BB, TOPK), lambda i: (i, 0)),
        pl.BlockSpec((BB, TOPK), lambda i: (i, 0)),
    )
    return pl.pallas_call(
        _tc_body,
        grid=grid,
        in_specs=[zb_spec, ab_spec, ret_spec, wqd_spec, wk_spec, bd_spec],
        out_specs=out_specs,
        out_shape=out_shapes,
    )(z, achs, ret2d, Wqd, Wk, bd2d)


def kernel(z, achs, anchor_att_ret, Wq, Wk, Wd, bd):
    ret2d = anchor_att_ret.reshape(B, K)
    bd2d = bd.reshape(1, D)
    Wqd = jnp.concatenate([Wq, Wd], axis=1)
    z_out, att, idx = _run(z, achs, ret2d, Wqd, Wk, bd2d)
    return (z_out, att.reshape(B, TOPK, 1), idx)


# speedup vs baseline: 4.2177x; 1.1124x over previous
import functools

import jax
import jax.numpy as jnp
from jax import lax
from jax.experimental import pallas as pl
from jax.experimental.pallas import tpu as pltpu

B, K, A, D = 128, 20, 64, 512
TOPK = K
BB = 64

_NEG = -3.0e38


def _tc_body(z_ref, achs_ref, ret_ref, wqd_ref, wk_ref, bd_ref,
             zout_ref, att_ref, idx_ref):
    zb = z_ref[...]
    ab = achs_ref[...]
    wqd = wqd_ref[...]
    wk = wk_ref[...]
    bd = bd_ref[...]

    qd = jnp.dot(ab.reshape(BB * A, D), wqd,
                 preferred_element_type=jnp.float32)
    q = qd[:, :D].reshape(BB, A, D)
    disp = (qd[:, D:] + bd).reshape(BB, A, D)
    kk = jnp.dot(zb.reshape(BB * K, D), wk,
                 preferred_element_type=jnp.float32).reshape(BB, K, D)

    scale = 1.0 / (D ** 0.5)
    logits = lax.dot_general(q, kk, (((2,), (2,)), ((0,), (0,))),
                             preferred_element_type=jnp.float32) * scale

    m = jnp.max(logits, axis=2, keepdims=True)
    lse = jnp.log(jnp.sum(jnp.exp(logits - m), axis=2, keepdims=True)) + m
    att_log = logits - lse

    ret = ret_ref[...]
    y = (att_log + ret[:, None, :]).reshape(BB, A * K)

    iota = lax.broadcasted_iota(jnp.int32, (BB, A * K), 1)
    vals = []
    idxs = []
    x = y
    for t in range(TOPK):
        mx = jnp.max(x, axis=1, keepdims=True)
        im = jnp.min(jnp.where(x == mx, iota, 10_000_000), axis=1,
                     keepdims=True)
        vals.append(mx)
        idxs.append(im)
        if t + 1 < TOPK:
            x = jnp.where(iota == im, _NEG, x)
    val = jnp.concatenate(vals, axis=1)
    fidx = jnp.concatenate(idxs, axis=1)

    a_idx = fidx // K
    k_idx = fidx - a_idx * K
    idx_ref[...] = k_idx * A + a_idx

    iota_ka = lax.broadcasted_iota(jnp.int32, (TOPK, K + A), 1)
    comb_idx = jnp.concatenate(
        [k_idx[:, :, None], (a_idx + K)[:, :, None]], axis=2)
    oh = ((comb_idx[:, :, 0:1] == iota_ka[None])
          | (comb_idx[:, :, 1:2] == iota_ka[None])).astype(jnp.float32)
    cat = jnp.concatenate([zb, disp], axis=1)
    znew = lax.dot_general(oh, cat, (((2,), (1,)), ((0,), (0,))),
                           preferred_element_type=jnp.float32)

    mu = jnp.mean(znew, axis=2, keepdims=True)
    var = jnp.mean((znew - mu) ** 2, axis=2, keepdims=True)
    zout_ref[...] = (znew - mu) / jnp.sqrt(var + 1e-5)

    att_ref[...] = val - val[:, 0:1]


@jax.jit
def _run(z, achs, ret2d, Wqd, Wk, bd2d):
    grid = (B // BB,)
    out_shapes = (
        jax.ShapeDtypeStruct((B, TOPK, D), jnp.float32),
        jax.ShapeDtypeStruct((B, TOPK), jnp.float32),
        jax.ShapeDtypeStruct((B, TOPK), jnp.int32),
    )
    zb_spec = pl.BlockSpec((BB, K, D), lambda i: (i, 0, 0))
    ab_spec = pl.BlockSpec((BB, A, D), lambda i: (i, 0, 0))
    ret_spec = pl.BlockSpec((BB, K), lambda i: (i, 0))
    wqd_spec = pl.BlockSpec((D, 2 * D), lambda i: (0, 0))
    wk_spec = pl.BlockSpec((D, D), lambda i: (0, 0))
    bd_spec = pl.BlockSpec((1, D), lambda i: (0, 0))
    out_specs = (
        pl.BlockSpec((BB, TOPK, D), lambda i: (i, 0, 0)),
        pl.BlockSpec((BB, TOPK), lambda i: (i, 0)),
        pl.BlockSpec((BB, TOPK), lambda i: (i, 0)),
    )
    return pl.pallas_call(
        _tc_body,
        grid=grid,
        in_specs=[zb_spec, ab_spec, ret_spec, wqd_spec, wk_spec, bd_spec],
        out_specs=out_specs,
        out_shape=out_shapes,
    )(z, achs, ret2d, Wqd, Wk, bd2d)


def kernel(z, achs, anchor_att_ret, Wq, Wk, Wd, bd):
    ret2d = anchor_att_ret.reshape(B, K)
    bd2d = bd.reshape(1, D)
    Wqd = jnp.concatenate([Wq, Wd], axis=1)
    z_out, att, idx = _run(z, achs, ret2d, Wqd, Wk, bd2d)
    return (z_out, att.reshape(B, TOPK, 1), idx)
